# Initial kernel scaffold; baseline (speedup 1.0000x reference)
import functools

import jax
import jax.numpy as jnp
from jax import lax
from jax.experimental import pallas as pl
from jax.experimental.pallas import tpu as pltpu
from jax.experimental.pallas import tpu_sc as plsc

B, S, T, D = 1024, 200, 8, 64
BS = B * S
NC, NS = 2, 16
NW = NC * NS
TPW = BS // NW
C = 128
NCHUNK = TPW // C
K = 16


def _base_body(feat_ref, w_ref, out_ref):
    out_ref[...] = jnp.dot(feat_ref[...], w_ref[...],
                           preferred_element_type=jnp.float32)


def _dense_base(feat, wcat):
    blk = 2048
    return pl.pallas_call(
        _base_body,
        grid=(BS // blk,),
        in_specs=[
            pl.BlockSpec((blk, K), lambda i: (i, 0)),
            pl.BlockSpec((K, D), lambda i: (0, 0)),
        ],
        out_specs=pl.BlockSpec((blk, D), lambda i: (i, 0)),
        out_shape=jax.ShapeDtypeStruct((BS, D), jnp.float32),
    )(feat, wcat)


def _sc_body(base_hbm, sp_ids, ev_ids, bu_ids, eq_ids, po_ids,
             sp_tab, ev_tab, bu_tab, eq_tab, po_tab,
             out_hbm,
             idx_v, rows_v, acc_v, sem):
    wid = lax.axis_index("s") * NC + lax.axis_index("c")
    wbase = wid * TPW

    def chunk(g, _):
        tok0 = wbase + g * C
        for t, ids in enumerate((sp_ids, ev_ids, bu_ids, eq_ids, po_ids)):
            pltpu.sync_copy(ids.at[pl.ds(tok0, C)], idx_v.at[t])
        descs = []
        for t, tab in enumerate((sp_tab, ev_tab, bu_tab, eq_tab, po_tab)):
            descs.append(pltpu.async_copy(tab.at[idx_v.at[t]], rows_v.at[t], sem))
        pltpu.sync_copy(base_hbm.at[pl.ds(tok0, C)], acc_v)
        for d in descs:
            d.wait()

        def tok(i, _):
            for j in range(D // 16):
                sl = pl.ds(j * 16, 16)
                v = acc_v[i, sl]
                v = v + rows_v[0, i, sl]
                v = v + rows_v[1, i, sl]
                v = v + rows_v[2, i, sl]
                v = v + rows_v[3, i, sl]
                v = v + rows_v[4, i, sl]
                acc_v[i, sl] = v
            return 0

        lax.fori_loop(0, C, tok, 0, unroll=2)
        pltpu.sync_copy(acc_v, out_hbm.at[pl.ds(tok0, C)])
        return 0

    lax.fori_loop(0, NCHUNK, chunk, 0)


def _gather_sum(base, sp_i, ev_i, bu_i, eq_i, po_i,
                sp_t, ev_t, bu_t, eq_t, po_t):
    mesh = plsc.VectorSubcoreMesh(core_axis_name="c", subcore_axis_name="s",
                                  num_cores=NC, num_subcores=NS)
    kern = pl.kernel(
        _sc_body,
        out_type=jax.ShapeDtypeStruct((BS, D), jnp.float32),
        mesh=mesh,
        scratch_types=[
            pltpu.VMEM((5, C), jnp.int32),
            pltpu.VMEM((5, C, D), jnp.float32),
            pltpu.VMEM((C, D), jnp.float32),
            pltpu.SemaphoreType.DMA,
        ],
    )
    return kern(base, sp_i, ev_i, bu_i, eq_i, po_i,
                sp_t, ev_t, bu_t, eq_t, po_t)


def kernel(time_features, spatial_ids, population, event_counts, event_types,
           building_counts, building_type_ids, equipment_ids, positions,
           W_time, b_time, spatial_table, W_pop, b_pop, W_ec, b_ec,
           event_type_table, W_bc, b_bc, building_type_table,
           equipment_table, position_table):
    f32 = jnp.float32
    feat = jnp.concatenate(
        [time_features.reshape(BS, T).astype(f32),
         population.reshape(BS, 1).astype(f32),
         event_counts.reshape(BS, 1).astype(f32),
         building_counts.reshape(BS, 1).astype(f32),
         jnp.ones((BS, 1), f32),
         jnp.zeros((BS, K - T - 4), f32)], axis=1)
    bias = (b_time + b_pop + b_ec + b_bc).astype(f32)
    wcat = jnp.concatenate(
        [W_time.astype(f32), W_pop.astype(f32), W_ec.astype(f32),
         W_bc.astype(f32), bias[None, :],
         jnp.zeros((K - T - 4, D), f32)], axis=0)

    base = _dense_base(feat, wcat)

    out = _gather_sum(
        base,
        spatial_ids.reshape(BS).astype(jnp.int32),
        event_types.reshape(BS).astype(jnp.int32),
        building_type_ids.reshape(BS).astype(jnp.int32),
        equipment_ids.reshape(BS).astype(jnp.int32),
        positions.reshape(BS).astype(jnp.int32),
        spatial_table.astype(f32), event_type_table.astype(f32),
        building_type_table.astype(f32), equipment_table.astype(f32),
        position_table.astype(f32))
    return out.reshape(B, S, D)


# speedup vs baseline: 4.1056x; 4.1056x over previous
import functools

import jax
import jax.numpy as jnp
from jax import lax
from jax.experimental import pallas as pl
from jax.experimental.pallas import tpu as pltpu
from jax.experimental.pallas import tpu_sc as plsc

B, S, T, D = 1024, 200, 8, 64
BS = B * S
NC, NS = 2, 16
NW = NC * NS
TPW = BS // NW
C = 128
NCHUNK = TPW // C
K = 16


def _base_body(feat_ref, w_ref, out_ref):
    out_ref[...] = jnp.dot(feat_ref[...], w_ref[...],
                           preferred_element_type=jnp.float32)


def _dense_base(feat, wcat):
    blk = 2048
    return pl.pallas_call(
        _base_body,
        grid=(BS // blk,),
        in_specs=[
            pl.BlockSpec((blk, K), lambda i: (i, 0)),
            pl.BlockSpec((K, D), lambda i: (0, 0)),
        ],
        out_specs=pl.BlockSpec((blk, D), lambda i: (i, 0)),
        out_shape=jax.ShapeDtypeStruct((BS, D), jnp.float32),
    )(feat, wcat)


def _sc_body(base_hbm, sp_ids, ev_ids, bu_ids, eq_ids, po_ids,
             sp_tab, ev_tab, bu_tab, eq_tab, po_tab,
             out_hbm,
             idx_v, rows_v, acc_v, sem):
    wid = lax.axis_index("s") * NC + lax.axis_index("c")
    wbase = wid * TPW

    def chunk(g, _):
        tok0 = wbase + g * C
        for t, ids in enumerate((sp_ids, ev_ids, bu_ids, eq_ids, po_ids)):
            pltpu.sync_copy(ids.at[pl.ds(tok0, C)], idx_v.at[t])
        descs = []
        for t, tab in enumerate((sp_tab, ev_tab, bu_tab, eq_tab, po_tab)):
            descs.append(pltpu.async_copy(tab.at[idx_v.at[t]], rows_v.at[t], sem))
        pltpu.sync_copy(base_hbm.at[pl.ds(tok0, C)], acc_v)
        for d in descs:
            d.wait()

        def tok(i, _):
            for j in range(D // 16):
                sl = pl.ds(j * 16, 16)
                v = acc_v[i, sl]
                v = v + rows_v[0, i, sl]
                v = v + rows_v[1, i, sl]
                v = v + rows_v[2, i, sl]
                v = v + rows_v[3, i, sl]
                v = v + rows_v[4, i, sl]
                acc_v[i, sl] = v
            return 0

        lax.fori_loop(0, C, tok, 0, unroll=2)
        pltpu.sync_copy(acc_v, out_hbm.at[pl.ds(tok0, C)])
        return 0

    lax.fori_loop(0, NCHUNK, chunk, 0)


def _gather_sum(base, sp_i, ev_i, bu_i, eq_i, po_i,
                sp_t, ev_t, bu_t, eq_t, po_t):
    mesh = plsc.VectorSubcoreMesh(core_axis_name="c", subcore_axis_name="s",
                                  num_cores=NC, num_subcores=NS)
    kern = pl.kernel(
        _sc_body,
        out_type=jax.ShapeDtypeStruct((BS, D), jnp.float32),
        mesh=mesh,
        scratch_types=[
            pltpu.VMEM((5, C), jnp.int32),
            pltpu.VMEM((5, C, D), jnp.float32),
            pltpu.VMEM((C, D), jnp.float32),
            pltpu.SemaphoreType.DMA,
        ],
        compiler_params=pltpu.CompilerParams(use_tc_tiling_on_sc=False),
    )
    return kern(base, sp_i, ev_i, bu_i, eq_i, po_i,
                sp_t, ev_t, bu_t, eq_t, po_t)


def kernel(time_features, spatial_ids, population, event_counts, event_types,
           building_counts, building_type_ids, equipment_ids, positions,
           W_time, b_time, spatial_table, W_pop, b_pop, W_ec, b_ec,
           event_type_table, W_bc, b_bc, building_type_table,
           equipment_table, position_table):
    f32 = jnp.float32
    feat = jnp.concatenate(
        [time_features.reshape(BS, T).astype(f32),
         population.reshape(BS, 1).astype(f32),
         event_counts.reshape(BS, 1).astype(f32),
         building_counts.reshape(BS, 1).astype(f32),
         jnp.ones((BS, 1), f32),
         jnp.zeros((BS, K - T - 4), f32)], axis=1)
    bias = (b_time + b_pop + b_ec + b_bc).astype(f32)
    wcat = jnp.concatenate(
        [W_time.astype(f32), W_pop.astype(f32), W_ec.astype(f32),
         W_bc.astype(f32), bias[None, :],
         jnp.zeros((K - T - 4, D), f32)], axis=0)

    base = _dense_base(feat, wcat)

    out = _gather_sum(
        base,
        spatial_ids.reshape(BS).astype(jnp.int32),
        event_types.reshape(BS).astype(jnp.int32),
        building_type_ids.reshape(BS).astype(jnp.int32),
        equipment_ids.reshape(BS).astype(jnp.int32),
        positions.reshape(BS).astype(jnp.int32),
        spatial_table.astype(f32), event_type_table.astype(f32),
        building_type_table.astype(f32), equipment_table.astype(f32),
        position_table.astype(f32))
    return out.reshape(B, S, D)


# speedup vs baseline: 5.5694x; 1.3565x over previous
import jax
import jax.numpy as jnp
from jax import lax
from jax.experimental import pallas as pl
from jax.experimental.pallas import tpu as pltpu
from jax.experimental.pallas import tpu_sc as plsc

B, S, T, D = 1024, 200, 8, 64
BS = B * S
NC, NS = 2, 16
NW = NC * NS
TPW = BS // NW
C = 128
NCHUNK = TPW // C
BB = 64


def _base_body(tf_ref, pop_ref, ec_ref, bc_ref, wt_ref, wp_ref, we_ref,
               wb_ref, bias_ref, out_ref):
    dense = lax.dot_general(tf_ref[...], wt_ref[...], (((2,), (0,)), ((), ())),
                            preferred_element_type=jnp.float32)
    dense += pop_ref[...][..., None] * wp_ref[...][0]
    dense += ec_ref[...][..., None] * we_ref[...][0]
    dense += bc_ref[...][..., None] * wb_ref[...][0]
    dense += bias_ref[...][0]
    out_ref[...] = dense


def _dense_base(tf, pop, ec, bc, wt, wp, we, wb, bias):
    grid = (B // BB,)
    return pl.pallas_call(
        _base_body,
        grid=grid,
        in_specs=[
            pl.BlockSpec((BB, S, T), lambda i: (i, 0, 0)),
            pl.BlockSpec((BB, S), lambda i: (i, 0)),
            pl.BlockSpec((BB, S), lambda i: (i, 0)),
            pl.BlockSpec((BB, S), lambda i: (i, 0)),
            pl.BlockSpec((T, D), lambda i: (0, 0)),
            pl.BlockSpec((1, D), lambda i: (0, 0)),
            pl.BlockSpec((1, D), lambda i: (0, 0)),
            pl.BlockSpec((1, D), lambda i: (0, 0)),
            pl.BlockSpec((1, D), lambda i: (0, 0)),
        ],
        out_specs=pl.BlockSpec((BB, S, D), lambda i: (i, 0, 0)),
        out_shape=jax.ShapeDtypeStruct((B, S, D), jnp.float32),
    )(tf, pop, ec, bc, wt, wp, we, wb, bias)


def _sc_body(base_hbm, sp_ids, ev_ids, bu_ids, eq_ids, po_ids,
             sp_tab, ev_tab, bu_tab, eq_tab, po_tab,
             out_hbm,
             idx_v, acc_v, sem_in, sem_g):
    wid = lax.axis_index("s") * NC + lax.axis_index("c")
    wbase = wid * TPW

    def chunk(g, _):
        tok0 = wbase + g * C
        ids_all = (sp_ids, ev_ids, bu_ids, eq_ids, po_ids)
        d_in = [pltpu.async_copy(ids.at[pl.ds(tok0, C)], idx_v.at[t], sem_in)
                for t, ids in enumerate(ids_all)]
        d_in.append(pltpu.async_copy(
            base_hbm.at[pl.ds(tok0, C)], acc_v, sem_in))
        for dsc in d_in:
            dsc.wait()

        tabs = (sp_tab, ev_tab, bu_tab, eq_tab, po_tab)
        d_g = [pltpu.async_copy(tab.at[idx_v.at[t]], acc_v, sem_g, add=True)
               for t, tab in enumerate(tabs)]
        for dsc in d_g:
            dsc.wait()

        pltpu.sync_copy(acc_v, out_hbm.at[pl.ds(tok0, C)])
        return 0

    lax.fori_loop(0, NCHUNK, chunk, 0)


def _sc_run(base, sp_i, ev_i, bu_i, eq_i, po_i,
            sp_t, ev_t, bu_t, eq_t, po_t):
    mesh = plsc.VectorSubcoreMesh(core_axis_name="c", subcore_axis_name="s",
                                  num_cores=NC, num_subcores=NS)
    kern = pl.kernel(
        _sc_body,
        out_type=jax.ShapeDtypeStruct((BS, D), jnp.float32),
        mesh=mesh,
        scratch_types=[
            pltpu.VMEM((5, C), jnp.int32),
            pltpu.VMEM((C, D), jnp.float32),
            pltpu.SemaphoreType.DMA,
            pltpu.SemaphoreType.DMA,
        ],
        compiler_params=pltpu.CompilerParams(use_tc_tiling_on_sc=False),
    )
    return kern(base, sp_i, ev_i, bu_i, eq_i, po_i,
                sp_t, ev_t, bu_t, eq_t, po_t)


def kernel(time_features, spatial_ids, population, event_counts, event_types,
           building_counts, building_type_ids, equipment_ids, positions,
           W_time, b_time, spatial_table, W_pop, b_pop, W_ec, b_ec,
           event_type_table, W_bc, b_bc, building_type_table,
           equipment_table, position_table):
    f32 = jnp.float32
    bias = (b_time + b_pop + b_ec + b_bc).astype(f32)
    base = _dense_base(
        time_features.astype(f32), population.astype(f32),
        event_counts.astype(f32), building_counts.astype(f32),
        W_time.astype(f32), W_pop.astype(f32), W_ec.astype(f32),
        W_bc.astype(f32), bias[None, :])

    out = _sc_run(
        base.reshape(BS, D),
        spatial_ids.reshape(BS).astype(jnp.int32),
        event_types.reshape(BS).astype(jnp.int32),
        building_type_ids.reshape(BS).astype(jnp.int32),
        equipment_ids.reshape(BS).astype(jnp.int32),
        positions.reshape(BS).astype(jnp.int32),
        spatial_table.astype(f32), event_type_table.astype(f32),
        building_type_table.astype(f32), equipment_table.astype(f32),
        position_table.astype(f32))
    return out.reshape(B, S, D)
